# baseline (speedup 1.0000x reference)
import jax
import jax.numpy as jnp
from jax.experimental import pallas as pl


_R = 256


def _body(x_ref, logits_ref, latent_ref):
    x = x_ref[...]
    s = jnp.sum(x * x, axis=1)
    logits_ref[...] = jnp.sqrt(s)
    m = jnp.max(s, axis=1, keepdims=True)
    d = s.shape[1]
    col = jax.lax.broadcasted_iota(jnp.int32, s.shape, 1)
    first = jnp.min(jnp.where(s == m, col, d), axis=1, keepdims=True)
    mask = (col == first).astype(x.dtype)
    latent_ref[...] = x * mask[:, None, :]


def kernel(x):
    B, C, D = x.shape
    grid = B // _R
    logits, latent = pl.pallas_call(
        _body,
        grid=(grid,),
        in_specs=[pl.BlockSpec((_R, C, D), lambda i: (i, 0, 0))],
        out_specs=[
            pl.BlockSpec((_R, D), lambda i: (i, 0)),
            pl.BlockSpec((_R, C, D), lambda i: (i, 0, 0)),
        ],
        out_shape=[
            jax.ShapeDtypeStruct((B, D), x.dtype),
            jax.ShapeDtypeStruct((B, C, D), x.dtype),
        ],
    )(x)
    return (logits, latent.reshape(B, C * D))
